# Initial kernel scaffold; baseline (speedup 1.0000x reference)
import jax
import jax.numpy as jnp
from jax.experimental import pallas as pl


def kernel(feat, edge_index, W1, b1):
    raise NotImplementedError("write your pallas kernel here")


# speedup vs baseline: 2.7236x; 2.7236x over previous
import functools

import jax
import jax.numpy as jnp
from jax import lax
from jax.experimental import pallas as pl
from jax.experimental.pallas import tpu as pltpu
from jax.experimental.pallas import tpu_sc as plsc

N = 10000
E = 320000
D = 128
NC = 2
NS = 16
NW = NC * NS
LANES = 16
BLK = 128
EPT = 10240
NBLK = EPT // BLK
EP = NW * EPT
NPAD = 10240
RPS = NPAD // NS
ICH = 8

_mesh = plsc.VectorSubcoreMesh(
    core_axis_name="c", subcore_axis_name="s", num_cores=NC, num_subcores=NS
)


def _fill(buf, rows, value):
    cols = buf.shape[1] // LANES

    @pl.loop(0, rows)
    def _(i):
        @pl.loop(0, cols)
        def _(j):
            buf[i, pl.ds(j * LANES, LANES)] = jnp.full((LANES,), value, jnp.float32)


@functools.partial(
    pl.kernel,
    out_type=jax.ShapeDtypeStruct((NC, NPAD, D), jnp.float32),
    mesh=_mesh,
    scratch_types=[
        pltpu.VMEM((ICH, BLK), jnp.int32),
        pltpu.VMEM((BLK, D), jnp.float32),
        pltpu.VMEM_SHARED((NPAD, D), jnp.float32),
    ],
)
def _deg_sc(dstr_hbm, out_hbm, dstv, ones, accum):
    c = lax.axis_index("c")
    s = lax.axis_index("s")
    wid = c * NS + s

    _fill(ones, BLK, 0.0)

    @pl.loop(0, RPS // BLK)
    def _(i):
        pltpu.sync_copy(ones, accum.at[pl.ds(s * RPS + i * BLK, BLK)])
    plsc.subcore_barrier()

    _fill(ones, BLK, 1.0)

    @pl.loop(0, NBLK // ICH)
    def _(k):
        pltpu.sync_copy(dstr_hbm.at[wid, pl.ds(k * ICH, ICH)], dstv)

        @pl.loop(0, ICH)
        def _(j):
            pltpu.sync_copy(ones, accum.at[dstv.at[j]], add=True)
    plsc.subcore_barrier()

    pltpu.sync_copy(accum.at[pl.ds(s * RPS, RPS)], out_hbm.at[c, pl.ds(s * RPS, RPS)])


@functools.partial(
    pl.kernel,
    out_type=jax.ShapeDtypeStruct((NC, NPAD, D), jnp.float32),
    mesh=_mesh,
    scratch_types=[
        pltpu.VMEM((ICH, BLK), jnp.int32),
        pltpu.VMEM((ICH, BLK), jnp.int32),
        pltpu.VMEM((BLK, D), jnp.float32),
        pltpu.VMEM_SHARED((NPAD, D), jnp.float32),
    ],
)
def _hop_sc(g_hbm, srcr_hbm, dstr_hbm, out_hbm, srcv, dstv, rows, accum):
    c = lax.axis_index("c")
    s = lax.axis_index("s")
    wid = c * NS + s

    _fill(rows, BLK, 0.0)

    @pl.loop(0, RPS // BLK)
    def _(i):
        pltpu.sync_copy(rows, accum.at[pl.ds(s * RPS + i * BLK, BLK)])
    plsc.subcore_barrier()

    @pl.loop(0, NBLK // ICH)
    def _(k):
        pltpu.sync_copy(srcr_hbm.at[wid, pl.ds(k * ICH, ICH)], srcv)
        pltpu.sync_copy(dstr_hbm.at[wid, pl.ds(k * ICH, ICH)], dstv)

        @pl.loop(0, ICH)
        def _(j):
            pltpu.sync_copy(g_hbm.at[srcv.at[j]], rows)
            pltpu.sync_copy(rows, accum.at[dstv.at[j]], add=True)
    plsc.subcore_barrier()

    pltpu.sync_copy(accum.at[pl.ds(s * RPS, RPS)], out_hbm.at[c, pl.ds(s * RPS, RPS)])


_RB = 1280
_GRID = NPAD // _RB

_deg_spec = pl.BlockSpec((NC, _RB, D), lambda i: (0, i, 0))
_row_spec = pl.BlockSpec((_RB, D), lambda i: (i, 0))
_par_spec = pl.BlockSpec((NC, _RB, D), lambda i: (0, i, 0))


def _norm_of(deg_ref):
    d = deg_ref[0, :, :1] + deg_ref[1, :, :1]
    return lax.rsqrt(jnp.maximum(d, 1.0))


def _scale_body(deg_ref, feat_ref, o_ref):
    o_ref[...] = feat_ref[...] * _norm_of(deg_ref)


_scale_call = pl.pallas_call(
    _scale_body,
    grid=(_GRID,),
    in_specs=[_deg_spec, _row_spec],
    out_specs=_row_spec,
    out_shape=jax.ShapeDtypeStruct((NPAD, D), jnp.float32),
)


def _comb_body(deg_ref, p_ref, o_ref):
    d = deg_ref[0, :, :1] + deg_ref[1, :, :1]
    o_ref[...] = (p_ref[0] + p_ref[1]) / jnp.maximum(d, 1.0)


_comb_call = pl.pallas_call(
    _comb_body,
    grid=(_GRID,),
    in_specs=[_deg_spec, _par_spec],
    out_specs=_row_spec,
    out_shape=jax.ShapeDtypeStruct((NPAD, D), jnp.float32),
)


def _final_body(deg_ref, p_ref, w_ref, b_ref, o_ref):
    h = (p_ref[0] + p_ref[1]) * _norm_of(deg_ref)
    o_ref[...] = (
        lax.dot_general(h, w_ref[...], (((1,), (1,)), ((), ())),
                        preferred_element_type=jnp.float32)
        + b_ref[...]
    )


_final_call = pl.pallas_call(
    _final_body,
    grid=(_GRID,),
    in_specs=[
        _deg_spec,
        _par_spec,
        pl.BlockSpec((D, D), lambda i: (0, 0)),
        pl.BlockSpec((1, D), lambda i: (0, 0)),
    ],
    out_specs=_row_spec,
    out_shape=jax.ShapeDtypeStruct((NPAD, D), jnp.float32),
)


def kernel(feat, edge_index, W1, b1):
    src = edge_index[0]
    dst = edge_index[1]
    pad = EP - E
    srcp = jnp.concatenate([src, jnp.zeros((pad,), jnp.int32)]).reshape(NW, NBLK, BLK)
    dstp = jnp.concatenate([dst, jnp.full((pad,), N, jnp.int32)]).reshape(NW, NBLK, BLK)
    featp = jnp.concatenate([feat, jnp.zeros((NPAD - N, D), feat.dtype)])

    degp = _deg_sc(dstp)
    g1 = _scale_call(degp, featp)
    p1 = _hop_sc(g1, srcp, dstp)
    g2 = _comb_call(degp, p1)
    p2 = _hop_sc(g2, srcp, dstp)
    x = _final_call(degp, p2, W1, b1.reshape(1, D))
    return x[:N]
